# Initial kernel scaffold; baseline (speedup 1.0000x reference)
import jax
import jax.numpy as jnp
from jax.experimental import pallas as pl


def kernel(x, edge_index, edge_attr, coords, bc_disp, bc_rot, Wn1, bn1, Wn2, bn2, We1, be1, We2, be2, Wmsg, bmsg, Wnode, bnode, Wd1, bd1, Wd2, bd2, Wd3, bd3, Wd4, bd4):
    raise NotImplementedError("write your pallas kernel here")


# speedup vs baseline: 3.4368x; 3.4368x over previous
import functools

import jax
import jax.numpy as jnp
from jax import lax
from jax.experimental import pallas as pl
from jax.experimental.pallas import tpu as pltpu
from jax.experimental.pallas import tpu_sc as plsc

N_NODES = 10000
N_EDGES = 320000
H = 128
N_LAYERS = 6

NC = 2
NS = 16
NW = NC * NS
EPT = N_EDGES // NW
CH = 80
NCHUNK = EPT // CH
N_PAD = 10240
RPT = N_PAD // NS
N_FULL = RPT // CH

BN = 2000
BE = 2560


def _mm2_body(x_ref, W1_ref, b1_ref, W2_ref, b2_ref, o_ref):
    z = jnp.maximum(
        jnp.dot(x_ref[...], W1_ref[...], preferred_element_type=jnp.float32)
        + b1_ref[...], 0.0)
    o_ref[...] = (jnp.dot(z, W2_ref[...], preferred_element_type=jnp.float32)
                  + b2_ref[...])


def _node_encode(x, Wn1, bn1, Wn2, bn2):
    k_in = x.shape[1]
    return pl.pallas_call(
        _mm2_body,
        grid=(N_NODES // BN,),
        in_specs=[
            pl.BlockSpec((BN, k_in), lambda i: (i, 0)),
            pl.BlockSpec((k_in, H), lambda i: (0, 0)),
            pl.BlockSpec((1, H), lambda i: (0, 0)),
            pl.BlockSpec((H, H), lambda i: (0, 0)),
            pl.BlockSpec((1, H), lambda i: (0, 0)),
        ],
        out_specs=pl.BlockSpec((BN, H), lambda i: (i, 0)),
        out_shape=jax.ShapeDtypeStruct((N_NODES, H), jnp.float32),
    )(x, Wn1, bn1, Wn2, bn2)


def _mm_relu_body(x_ref, W_ref, b_ref, o_ref):
    o_ref[...] = jnp.maximum(
        jnp.dot(x_ref[...], W_ref[...], preferred_element_type=jnp.float32)
        + b_ref[...], 0.0)


def _edge_t(edge_attr, We1, be1):
    k_in = edge_attr.shape[1]
    return pl.pallas_call(
        _mm_relu_body,
        grid=(N_EDGES // BE,),
        in_specs=[
            pl.BlockSpec((BE, k_in), lambda i: (i, 0)),
            pl.BlockSpec((k_in, H), lambda i: (0, 0)),
            pl.BlockSpec((1, H), lambda i: (0, 0)),
        ],
        out_specs=pl.BlockSpec((BE, H), lambda i: (i, 0)),
        out_shape=jax.ShapeDtypeStruct((N_EDGES, H), jnp.float32),
    )(edge_attr, We1, be1)


def _fold_body(We2_ref, be2_ref, Wc_ref, bm_ref, W2c_ref, cb_ref):
    Wc = Wc_ref[0]
    W2c_ref[0] = jnp.dot(We2_ref[...], Wc, preferred_element_type=jnp.float32)
    cb_ref[0] = (jnp.dot(be2_ref[...], Wc, preferred_element_type=jnp.float32)
                 + bm_ref[0])


def _fold_weights(We2, be2, Wc, bmsg3):
    return pl.pallas_call(
        _fold_body,
        grid=(N_LAYERS,),
        in_specs=[
            pl.BlockSpec((H, H), lambda l: (0, 0)),
            pl.BlockSpec((1, H), lambda l: (0, 0)),
            pl.BlockSpec((1, H, H), lambda l: (l, 0, 0)),
            pl.BlockSpec((1, 1, H), lambda l: (l, 0, 0)),
        ],
        out_specs=[
            pl.BlockSpec((1, H, H), lambda l: (l, 0, 0)),
            pl.BlockSpec((1, 1, H), lambda l: (l, 0, 0)),
        ],
        out_shape=[
            jax.ShapeDtypeStruct((N_LAYERS, H, H), jnp.float32),
            jax.ShapeDtypeStruct((N_LAYERS, 1, H), jnp.float32),
        ],
    )(We2, be2, Wc, bmsg3)


def _ab_body(h_ref, Wa_ref, Wb_ref, a_ref, b_ref):
    h = h_ref[...]
    a_ref[...] = jnp.dot(h, Wa_ref[...], preferred_element_type=jnp.float32)
    b_ref[...] = jnp.dot(h, Wb_ref[...], preferred_element_type=jnp.float32)


def _ab_tables(h, Wa, Wb):
    return pl.pallas_call(
        _ab_body,
        grid=(N_NODES // BN,),
        in_specs=[
            pl.BlockSpec((BN, H), lambda i: (i, 0)),
            pl.BlockSpec((H, H), lambda i: (0, 0)),
            pl.BlockSpec((H, H), lambda i: (0, 0)),
        ],
        out_specs=[
            pl.BlockSpec((BN, H), lambda i: (i, 0)),
            pl.BlockSpec((BN, H), lambda i: (i, 0)),
        ],
        out_shape=[
            jax.ShapeDtypeStruct((N_NODES, H), jnp.float32),
            jax.ShapeDtypeStruct((N_NODES, H), jnp.float32),
        ],
    )(h, Wa, Wb)


def _mm_bias_body(x_ref, W_ref, b_ref, o_ref):
    o_ref[...] = (jnp.dot(x_ref[...], W_ref[...],
                          preferred_element_type=jnp.float32) + b_ref[...])


def _edge_ec(t, W2c_l, cb_l):
    return pl.pallas_call(
        _mm_bias_body,
        grid=(N_EDGES // BE,),
        in_specs=[
            pl.BlockSpec((BE, H), lambda i: (i, 0)),
            pl.BlockSpec((H, H), lambda i: (0, 0)),
            pl.BlockSpec((1, H), lambda i: (0, 0)),
        ],
        out_specs=pl.BlockSpec((BE, H), lambda i: (i, 0)),
        out_shape=jax.ShapeDtypeStruct((N_EDGES, H), jnp.float32),
    )(t, W2c_l, cb_l)


def _node_update_body(h_ref, p_ref, Wnh_ref, Wna_ref, bn_ref, o_ref):
    h = h_ref[...]
    agg = p_ref[0] + p_ref[1]
    z = (jnp.dot(h, Wnh_ref[...], preferred_element_type=jnp.float32)
         + jnp.dot(agg, Wna_ref[...], preferred_element_type=jnp.float32)
         + bn_ref[...])
    o_ref[...] = h + jnp.maximum(z, 0.0)


def _node_update(h, parts, Wnh, Wna, bn):
    return pl.pallas_call(
        _node_update_body,
        grid=(N_NODES // BN,),
        in_specs=[
            pl.BlockSpec((BN, H), lambda i: (i, 0)),
            pl.BlockSpec((NC, BN, H), lambda i: (0, i, 0)),
            pl.BlockSpec((H, H), lambda i: (0, 0)),
            pl.BlockSpec((H, H), lambda i: (0, 0)),
            pl.BlockSpec((1, H), lambda i: (0, 0)),
        ],
        out_specs=pl.BlockSpec((BN, H), lambda i: (i, 0)),
        out_shape=jax.ShapeDtypeStruct((N_NODES, H), jnp.float32),
    )(h, parts, Wnh, Wna, bn)


def _decoder_body(c_ref, h_ref, bcd_ref, bcr_ref, W1c_ref, W1h_ref, b1_ref,
                  W2_ref, b2_ref, W3_ref, b3_ref, W4_ref, b4_ref, o_ref):
    z = jnp.maximum(
        jnp.dot(c_ref[...], W1c_ref[...], preferred_element_type=jnp.float32)
        + jnp.dot(h_ref[...], W1h_ref[...], preferred_element_type=jnp.float32)
        + b1_ref[...], 0.0)
    z = jnp.maximum(
        jnp.dot(z, W2_ref[...], preferred_element_type=jnp.float32)
        + b2_ref[...], 0.0)
    z = jnp.maximum(
        jnp.dot(z, W3_ref[...], preferred_element_type=jnp.float32)
        + b3_ref[...], 0.0)
    p = (jnp.dot(z, W4_ref[...], preferred_element_type=jnp.float32)
         + b4_ref[...])
    col = lax.broadcasted_iota(jnp.int32, p.shape, 1)
    scale = jnp.where(col < 2, 1.0 - bcd_ref[...], 1.0 - bcr_ref[...])
    o_ref[...] = p * scale


def _decode(coords, h, bc_disp, bc_rot, Wd1c, Wd1h, bd1, Wd2, bd2,
            Wd3, bd3, Wd4, bd4):
    out_dim = Wd4.shape[1]
    return pl.pallas_call(
        _decoder_body,
        grid=(N_NODES // BN,),
        in_specs=[
            pl.BlockSpec((BN, 3), lambda i: (i, 0)),
            pl.BlockSpec((BN, H), lambda i: (i, 0)),
            pl.BlockSpec((BN, 1), lambda i: (i, 0)),
            pl.BlockSpec((BN, 1), lambda i: (i, 0)),
            pl.BlockSpec((3, H), lambda i: (0, 0)),
            pl.BlockSpec((H, H), lambda i: (0, 0)),
            pl.BlockSpec((1, H), lambda i: (0, 0)),
            pl.BlockSpec((H, H), lambda i: (0, 0)),
            pl.BlockSpec((1, H), lambda i: (0, 0)),
            pl.BlockSpec((H, 64), lambda i: (0, 0)),
            pl.BlockSpec((1, 64), lambda i: (0, 0)),
            pl.BlockSpec((64, out_dim), lambda i: (0, 0)),
            pl.BlockSpec((1, out_dim), lambda i: (0, 0)),
        ],
        out_specs=pl.BlockSpec((BN, out_dim), lambda i: (i, 0)),
        out_shape=jax.ShapeDtypeStruct((N_NODES, out_dim), jnp.float32),
    )(coords, h, bc_disp, bc_rot, Wd1c, Wd1h, bd1, Wd2, bd2, Wd3, bd3,
      Wd4, bd4)


@functools.lru_cache(maxsize=1)
def _build_sc_agg():
    mesh = plsc.VectorSubcoreMesh(core_axis_name="c", subcore_axis_name="s",
                                  num_cores=NC, num_subcores=NS)

    @functools.partial(
        pl.kernel,
        out_type=jax.ShapeDtypeStruct((NC, N_PAD, H), jnp.float32),
        mesh=mesh,
        scratch_types=[
            pltpu.VMEM((CH,), jnp.int32),
            pltpu.VMEM((CH,), jnp.int32),
            pltpu.VMEM((CH, H), jnp.float32),
            pltpu.VMEM((CH, H), jnp.float32),
            pltpu.VMEM((CH, H), jnp.float32),
            pltpu.VMEM_SHARED((N_PAD, H), jnp.float32),
            pltpu.SemaphoreType.DMA,
            pltpu.SemaphoreType.DMA,
            pltpu.SemaphoreType.DMA,
        ],
    )
    def sc_agg(a_hbm, b_hbm, ec_hbm, src_hbm, dst_hbm, out_hbm,
               src_v, dst_v, a_v, b_v, ec_v, agg_sh, sem_a, sem_b, sem_e):
        cid = lax.axis_index("c")
        sid = lax.axis_index("s")

        def _zero_row(r, carry):
            for j in range(H // 16):
                a_v[r, pl.ds(j * 16, 16)] = jnp.zeros((16,), jnp.float32)
            return carry
        lax.fori_loop(0, CH, _zero_row, 0)

        row0 = sid * RPT

        def _zero_cp(k, carry):
            pltpu.sync_copy(a_v, agg_sh.at[pl.ds(row0 + k * CH, CH)])
            return carry
        lax.fori_loop(0, N_FULL, _zero_cp, 0)
        plsc.subcore_barrier()

        ebase = (cid * NS + sid) * EPT

        def _chunk(c, carry):
            e0 = ebase + c * CH
            pltpu.sync_copy(src_hbm.at[pl.ds(e0, CH)], src_v)
            pltpu.sync_copy(dst_hbm.at[pl.ds(e0, CH)], dst_v)
            cp_a = pltpu.async_copy(a_hbm.at[src_v], a_v, sem_a)
            cp_b = pltpu.async_copy(b_hbm.at[dst_v], b_v, sem_b)
            cp_e = pltpu.async_copy(ec_hbm.at[pl.ds(e0, CH)], ec_v, sem_e)
            cp_a.wait()
            cp_b.wait()
            cp_e.wait()

            def _row(r, rc):
                for j in range(H // 16):
                    s = pl.ds(j * 16, 16)
                    v = a_v[r, s] + b_v[r, s] + ec_v[r, s]
                    ec_v[r, s] = jnp.maximum(v, 0.0)
                return rc
            lax.fori_loop(0, CH, _row, 0)

            pltpu.sync_copy(ec_v, agg_sh.at[dst_v], add=True)
            return carry
        lax.fori_loop(0, NCHUNK, _chunk, 0)
        plsc.subcore_barrier()

        def _out_cp(k, carry):
            r0 = row0 + k * CH
            pltpu.sync_copy(agg_sh.at[pl.ds(r0, CH)], a_v)
            pltpu.sync_copy(a_v, out_hbm.at[cid, pl.ds(r0, CH)])
            return carry
        lax.fori_loop(0, N_FULL, _out_cp, 0)

    return sc_agg


def _sc_agg(a_tab, b_tab, ec, src, dst):
    return _build_sc_agg()(a_tab, b_tab, ec, src, dst)


def kernel(x, edge_index, edge_attr, coords, bc_disp, bc_rot,
           Wn1, bn1, Wn2, bn2, We1, be1, We2, be2,
           Wmsg, bmsg, Wnode, bnode,
           Wd1, bd1, Wd2, bd2, Wd3, bd3, Wd4, bd4):
    src = edge_index[0]
    dst = edge_index[1]

    Wa = Wmsg[:, :H, :]
    Wb = Wmsg[:, H:2 * H, :]
    Wc = Wmsg[:, 2 * H:, :]
    Wnh = Wnode[:, :H, :]
    Wna = Wnode[:, H:, :]

    h = _node_encode(x, Wn1, bn1.reshape(1, H), Wn2, bn2.reshape(1, H))
    t = _edge_t(edge_attr, We1, be1.reshape(1, H))
    W2c, cb = _fold_weights(We2, be2.reshape(1, H), Wc,
                            bmsg.reshape(N_LAYERS, 1, H))

    for l in range(N_LAYERS):
        a_tab, b_tab = _ab_tables(h, Wa[l], Wb[l])
        ec = _edge_ec(t, W2c[l], cb[l])
        parts = _sc_agg(a_tab, b_tab, ec, src, dst)
        h = _node_update(h, parts, Wnh[l], Wna[l],
                         bnode[l].reshape(1, H))

    pred = _decode(coords, h, bc_disp, bc_rot,
                   Wd1[:3], Wd1[3:], bd1.reshape(1, H),
                   Wd2, bd2.reshape(1, H),
                   Wd3, bd3.reshape(1, 64),
                   Wd4, bd4.reshape(1, Wd4.shape[1]))
    return pred


# speedup vs baseline: 4.0756x; 1.1858x over previous
import functools

import jax
import jax.numpy as jnp
from jax import lax
from jax.experimental import pallas as pl
from jax.experimental.pallas import tpu as pltpu
from jax.experimental.pallas import tpu_sc as plsc

N_NODES = 10000
N_EDGES = 320000
H = 128
N_LAYERS = 6

NC = 2
NS = 16
NW = NC * NS
N_PAD = 10240
RPT = N_PAD // NS

BN = 2000
BE = 2560


def _mm2_body(x_ref, W1_ref, b1_ref, W2_ref, b2_ref, o_ref):
    z = jnp.maximum(
        jnp.dot(x_ref[...], W1_ref[...], preferred_element_type=jnp.float32)
        + b1_ref[...], 0.0)
    o_ref[...] = (jnp.dot(z, W2_ref[...], preferred_element_type=jnp.float32)
                  + b2_ref[...])


def _node_encode(x, Wn1, bn1, Wn2, bn2):
    k_in = x.shape[1]
    return pl.pallas_call(
        _mm2_body,
        grid=(N_NODES // BN,),
        in_specs=[
            pl.BlockSpec((BN, k_in), lambda i: (i, 0)),
            pl.BlockSpec((k_in, H), lambda i: (0, 0)),
            pl.BlockSpec((1, H), lambda i: (0, 0)),
            pl.BlockSpec((H, H), lambda i: (0, 0)),
            pl.BlockSpec((1, H), lambda i: (0, 0)),
        ],
        out_specs=pl.BlockSpec((BN, H), lambda i: (i, 0)),
        out_shape=jax.ShapeDtypeStruct((N_NODES, H), jnp.float32),
    )(x, Wn1, bn1, Wn2, bn2)


def _mm_relu_body(x_ref, W_ref, b_ref, o_ref):
    o_ref[...] = jnp.maximum(
        jnp.dot(x_ref[...], W_ref[...], preferred_element_type=jnp.float32)
        + b_ref[...], 0.0)


def _edge_t(edge_attr, We1, be1):
    k_in = edge_attr.shape[1]
    return pl.pallas_call(
        _mm_relu_body,
        grid=(N_EDGES // BE,),
        in_specs=[
            pl.BlockSpec((BE, k_in), lambda i: (i, 0)),
            pl.BlockSpec((k_in, H), lambda i: (0, 0)),
            pl.BlockSpec((1, H), lambda i: (0, 0)),
        ],
        out_specs=pl.BlockSpec((BE, H), lambda i: (i, 0)),
        out_shape=jax.ShapeDtypeStruct((N_EDGES, H), jnp.float32),
    )(edge_attr, We1, be1)


def _fold_body(We2_ref, be2_ref, Wc_ref, bm_ref, W2c_ref, cb_ref):
    Wc = Wc_ref[0]
    W2c_ref[0] = jnp.dot(We2_ref[...], Wc, preferred_element_type=jnp.float32)
    cb_ref[0] = (jnp.dot(be2_ref[...], Wc, preferred_element_type=jnp.float32)
                 + bm_ref[0])


def _fold_weights(We2, be2, Wc, bmsg3):
    return pl.pallas_call(
        _fold_body,
        grid=(N_LAYERS,),
        in_specs=[
            pl.BlockSpec((H, H), lambda l: (0, 0)),
            pl.BlockSpec((1, H), lambda l: (0, 0)),
            pl.BlockSpec((1, H, H), lambda l: (l, 0, 0)),
            pl.BlockSpec((1, 1, H), lambda l: (l, 0, 0)),
        ],
        out_specs=[
            pl.BlockSpec((1, H, H), lambda l: (l, 0, 0)),
            pl.BlockSpec((1, 1, H), lambda l: (l, 0, 0)),
        ],
        out_shape=[
            jax.ShapeDtypeStruct((N_LAYERS, H, H), jnp.float32),
            jax.ShapeDtypeStruct((N_LAYERS, 1, H), jnp.float32),
        ],
    )(We2, be2, Wc, bmsg3)


def _ab_body(h_ref, Wa_ref, Wb_ref, a_ref, b_ref):
    h = h_ref[...]
    a_ref[...] = jnp.dot(h, Wa_ref[...], preferred_element_type=jnp.float32)
    b_ref[...] = jnp.dot(h, Wb_ref[...], preferred_element_type=jnp.float32)


def _ab_tables(h, Wa, Wb):
    return pl.pallas_call(
        _ab_body,
        grid=(N_NODES // BN,),
        in_specs=[
            pl.BlockSpec((BN, H), lambda i: (i, 0)),
            pl.BlockSpec((H, H), lambda i: (0, 0)),
            pl.BlockSpec((H, H), lambda i: (0, 0)),
        ],
        out_specs=[
            pl.BlockSpec((BN, H), lambda i: (i, 0)),
            pl.BlockSpec((BN, H), lambda i: (i, 0)),
        ],
        out_shape=[
            jax.ShapeDtypeStruct((N_NODES, H), jnp.float32),
            jax.ShapeDtypeStruct((N_NODES, H), jnp.float32),
        ],
    )(h, Wa, Wb)


def _mm_bias_body(x_ref, W_ref, b_ref, o_ref):
    o_ref[...] = (jnp.dot(x_ref[...], W_ref[...],
                          preferred_element_type=jnp.float32) + b_ref[...])


def _edge_ec(t, W2c_l, cb_l):
    return pl.pallas_call(
        _mm_bias_body,
        grid=(N_EDGES // BE,),
        in_specs=[
            pl.BlockSpec((BE, H), lambda i: (i, 0)),
            pl.BlockSpec((H, H), lambda i: (0, 0)),
            pl.BlockSpec((1, H), lambda i: (0, 0)),
        ],
        out_specs=pl.BlockSpec((BE, H), lambda i: (i, 0)),
        out_shape=jax.ShapeDtypeStruct((N_EDGES, H), jnp.float32),
    )(t, W2c_l, cb_l)


def _node_update_body(h_ref, p_ref, Wnh_ref, Wna_ref, bn_ref, o_ref):
    h = h_ref[...]
    agg = p_ref[0] + p_ref[1]
    z = (jnp.dot(h, Wnh_ref[...], preferred_element_type=jnp.float32)
         + jnp.dot(agg, Wna_ref[...], preferred_element_type=jnp.float32)
         + bn_ref[...])
    o_ref[...] = h + jnp.maximum(z, 0.0)


def _node_update(h, parts, Wnh, Wna, bn):
    return pl.pallas_call(
        _node_update_body,
        grid=(N_NODES // BN,),
        in_specs=[
            pl.BlockSpec((BN, H), lambda i: (i, 0)),
            pl.BlockSpec((NC, BN, H), lambda i: (0, i, 0)),
            pl.BlockSpec((H, H), lambda i: (0, 0)),
            pl.BlockSpec((H, H), lambda i: (0, 0)),
            pl.BlockSpec((1, H), lambda i: (0, 0)),
        ],
        out_specs=pl.BlockSpec((BN, H), lambda i: (i, 0)),
        out_shape=jax.ShapeDtypeStruct((N_NODES, H), jnp.float32),
    )(h, parts, Wnh, Wna, bn)


def _decoder_body(c_ref, h_ref, bcd_ref, bcr_ref, W1c_ref, W1h_ref, b1_ref,
                  W2_ref, b2_ref, W3_ref, b3_ref, W4_ref, b4_ref, o_ref):
    z = jnp.maximum(
        jnp.dot(c_ref[...], W1c_ref[...], preferred_element_type=jnp.float32)
        + jnp.dot(h_ref[...], W1h_ref[...], preferred_element_type=jnp.float32)
        + b1_ref[...], 0.0)
    z = jnp.maximum(
        jnp.dot(z, W2_ref[...], preferred_element_type=jnp.float32)
        + b2_ref[...], 0.0)
    z = jnp.maximum(
        jnp.dot(z, W3_ref[...], preferred_element_type=jnp.float32)
        + b3_ref[...], 0.0)
    p = (jnp.dot(z, W4_ref[...], preferred_element_type=jnp.float32)
         + b4_ref[...])
    col = lax.broadcasted_iota(jnp.int32, p.shape, 1)
    scale = jnp.where(col < 2, 1.0 - bcd_ref[...], 1.0 - bcr_ref[...])
    o_ref[...] = p * scale


def _decode(coords, h, bc_disp, bc_rot, Wd1c, Wd1h, bd1, Wd2, bd2,
            Wd3, bd3, Wd4, bd4):
    out_dim = Wd4.shape[1]
    return pl.pallas_call(
        _decoder_body,
        grid=(N_NODES // BN,),
        in_specs=[
            pl.BlockSpec((BN, 3), lambda i: (i, 0)),
            pl.BlockSpec((BN, H), lambda i: (i, 0)),
            pl.BlockSpec((BN, 1), lambda i: (i, 0)),
            pl.BlockSpec((BN, 1), lambda i: (i, 0)),
            pl.BlockSpec((3, H), lambda i: (0, 0)),
            pl.BlockSpec((H, H), lambda i: (0, 0)),
            pl.BlockSpec((1, H), lambda i: (0, 0)),
            pl.BlockSpec((H, H), lambda i: (0, 0)),
            pl.BlockSpec((1, H), lambda i: (0, 0)),
            pl.BlockSpec((H, 64), lambda i: (0, 0)),
            pl.BlockSpec((1, 64), lambda i: (0, 0)),
            pl.BlockSpec((64, out_dim), lambda i: (0, 0)),
            pl.BlockSpec((1, out_dim), lambda i: (0, 0)),
        ],
        out_specs=pl.BlockSpec((BN, out_dim), lambda i: (i, 0)),
        out_shape=jax.ShapeDtypeStruct((N_NODES, out_dim), jnp.float32),
    )(coords, h, bc_disp, bc_rot, Wd1c, Wd1h, bd1, Wd2, bd2, Wd3, bd3,
      Wd4, bd4)


CH = 32
NCH_BIG = 314
NCH_SMALL = 312
EB_SMALL0 = 8 * NCH_BIG * CH


@functools.lru_cache(maxsize=1)
def _build_sc_agg():
    mesh = plsc.VectorSubcoreMesh(core_axis_name="c", subcore_axis_name="s",
                                  num_cores=NC, num_subcores=NS)

    @functools.partial(
        pl.kernel,
        out_type=jax.ShapeDtypeStruct((NC, N_PAD, H), jnp.float32),
        mesh=mesh,
        scratch_types=[
            pltpu.VMEM((CH,), jnp.int32),
            pltpu.VMEM((CH,), jnp.int32),
            pltpu.VMEM((CH,), jnp.int32),
            pltpu.VMEM((CH,), jnp.int32),
            pltpu.VMEM((CH,), jnp.int32),
            pltpu.VMEM((CH,), jnp.int32),
            pltpu.VMEM((CH, H), jnp.float32),
            pltpu.VMEM((CH, H), jnp.float32),
            pltpu.VMEM((CH, H), jnp.float32),
            pltpu.VMEM((CH, H), jnp.float32),
            pltpu.VMEM((CH, H), jnp.float32),
            pltpu.VMEM((CH, H), jnp.float32),
            pltpu.VMEM((CH, H), jnp.float32),
            pltpu.VMEM((CH, H), jnp.float32),
            pltpu.VMEM_SHARED((N_PAD, H), jnp.float32),
        ] + [pltpu.SemaphoreType.DMA] * 8,
    )
    def sc_agg(a_hbm, b_hbm, ec_hbm, src_hbm, dst_hbm, out_hbm,
               si0, si1, di0, di1, ds0, ds1, a0, a1, b0, b1, e0, e1, m0, m1,
               agg_sh, sA0, sA1, sB0, sB1, sE0, sE1, sS0, sS1):
        cid = lax.axis_index("c")
        sid = lax.axis_index("s")
        wid = cid * NS + sid

        BUFS = ((si0, di0, ds0, a0, b0, e0, m0, sA0, sB0, sE0, sS0),
                (si1, di1, ds1, a1, b1, e1, m1, sA1, sB1, sE1, sS1))

        def _zero_row(r, carry):
            for j in range(H // 16):
                m0[r, pl.ds(j * 16, 16)] = jnp.zeros((16,), jnp.float32)
            return carry
        lax.fori_loop(0, CH, _zero_row, 0)

        row0 = sid * RPT

        def _zero_cp(k, carry):
            pltpu.sync_copy(m0, agg_sh.at[pl.ds(row0 + k * CH, CH)])
            return carry
        lax.fori_loop(0, RPT // CH, _zero_cp, 0)
        plsc.subcore_barrier()

        ebase = jnp.where(wid < 8, wid * (NCH_BIG * CH),
                          EB_SMALL0 + (wid - 8) * (NCH_SMALL * CH))
        nch = jnp.where(wid < 8, NCH_BIG, NCH_SMALL)

        def _issue(P, eo):
            six, dix, dsx, av, bv, ev, mv, sa, sb, se, ss = BUFS[P]
            pltpu.sync_copy(src_hbm.at[pl.ds(eo, CH)], six)
            pltpu.sync_copy(dst_hbm.at[pl.ds(eo, CH)], dix)
            pltpu.async_copy(a_hbm.at[six], av, sa)
            pltpu.async_copy(b_hbm.at[dix], bv, sb)
            pltpu.async_copy(ec_hbm.at[pl.ds(eo, CH)], ev, se)

        for P in range(2):
            _issue(P, ebase + P * CH)

        def _pair(c2, carry):
            for P in range(2):
                six, dix, dsx, av, bv, ev, mv, sa, sb, se, ss = BUFS[P]
                c = 2 * c2 + P
                eo = ebase + c * CH
                pltpu.make_async_copy(a_hbm.at[six], av, sa).wait()
                pltpu.make_async_copy(b_hbm.at[dix], bv, sb).wait()
                pltpu.make_async_copy(ec_hbm.at[pl.ds(eo, CH)], ev, se).wait()

                @pl.when(c2 > 0)
                def _():
                    pltpu.make_async_copy(mv, agg_sh.at[dsx], ss).wait()

                dsx[pl.ds(0, 16)] = dix[pl.ds(0, 16)]
                dsx[pl.ds(16, 16)] = dix[pl.ds(16, 16)]

                def _row(r, rc):
                    for j in range(H // 16):
                        s = pl.ds(j * 16, 16)
                        v = av[r, s] + bv[r, s] + ev[r, s]
                        mv[r, s] = jnp.maximum(v, 0.0)
                    return rc
                lax.fori_loop(0, CH, _row, 0)

                pltpu.async_copy(mv, agg_sh.at[dsx], ss, add=True)

                @pl.when(c + 2 < nch)
                def _():
                    _issue(P, eo + 2 * CH)
            return carry
        lax.fori_loop(0, nch // 2, _pair, 0)

        for P in range(2):
            six, dix, dsx, av, bv, ev, mv, sa, sb, se, ss = BUFS[P]
            pltpu.make_async_copy(mv, agg_sh.at[dsx], ss).wait()
        plsc.subcore_barrier()

        def _out_cp(k, carry):
            r0 = row0 + k * CH
            pltpu.sync_copy(agg_sh.at[pl.ds(r0, CH)], m0)
            pltpu.sync_copy(m0, out_hbm.at[cid, pl.ds(r0, CH)])
            return carry
        lax.fori_loop(0, RPT // CH, _out_cp, 0)

    return sc_agg


def _sc_agg(a_tab, b_tab, ec, src, dst):
    return _build_sc_agg()(a_tab, b_tab, ec, src, dst)


def kernel(x, edge_index, edge_attr, coords, bc_disp, bc_rot,
           Wn1, bn1, Wn2, bn2, We1, be1, We2, be2,
           Wmsg, bmsg, Wnode, bnode,
           Wd1, bd1, Wd2, bd2, Wd3, bd3, Wd4, bd4):
    src = edge_index[0]
    dst = edge_index[1]

    Wa = Wmsg[:, :H, :]
    Wb = Wmsg[:, H:2 * H, :]
    Wc = Wmsg[:, 2 * H:, :]
    Wnh = Wnode[:, :H, :]
    Wna = Wnode[:, H:, :]

    h = _node_encode(x, Wn1, bn1.reshape(1, H), Wn2, bn2.reshape(1, H))
    t = _edge_t(edge_attr, We1, be1.reshape(1, H))
    W2c, cb = _fold_weights(We2, be2.reshape(1, H), Wc,
                            bmsg.reshape(N_LAYERS, 1, H))

    for l in range(N_LAYERS):
        a_tab, b_tab = _ab_tables(h, Wa[l], Wb[l])
        ec = _edge_ec(t, W2c[l], cb[l])
        parts = _sc_agg(a_tab, b_tab, ec, src, dst)
        h = _node_update(h, parts, Wnh[l], Wna[l],
                         bnode[l].reshape(1, H))

    pred = _decode(coords, h, bc_disp, bc_rot,
                   Wd1[:3], Wd1[3:], bd1.reshape(1, H),
                   Wd2, bd2.reshape(1, H),
                   Wd3, bd3.reshape(1, 64),
                   Wd4, bd4.reshape(1, Wd4.shape[1]))
    return pred
